# baseline (speedup 1.0000x reference)
import functools

import jax
import jax.numpy as jnp
from jax import lax
from jax.experimental import pallas as pl
from jax.experimental.pallas import tpu as pltpu
from jax.experimental.pallas import tpu_sc as plsc

_D = 32
_BC = 32
_NW = 32
_ROT = 1


@functools.partial(jax.jit, static_argnums=(3, 4))
def _gather_weight(table, idxf, wf, n_b, n_l):
    bpw = n_b // _NW
    n_chunks = bpw // _BC
    chunk_rows = _BC * n_l
    rows_per_w = bpw * n_l
    mesh = plsc.VectorSubcoreMesh(core_axis_name="c", subcore_axis_name="s")

    @functools.partial(
        pl.kernel,
        mesh=mesh,
        out_type=jax.ShapeDtypeStruct((n_b, n_l, _D), jnp.float32),
        compiler_params=pltpu.CompilerParams(use_tc_tiling_on_sc=False),
        scratch_types=[
            pltpu.VMEM((rows_per_w,), jnp.int32),
            pltpu.VMEM((rows_per_w,), jnp.float32),
            pltpu.VMEM((chunk_rows, _D), jnp.float32),
            pltpu.SemaphoreType.DMA,
        ],
    )
    def k(table_hbm, idx_hbm, w_hbm, out_hbm, idxf_v, wf_v, rows_v, sem):
        wid = lax.axis_index("s") * 2 + lax.axis_index("c")
        b0_out = lax.rem(wid + _ROT, _NW) * bpw

        pltpu.sync_copy(idx_hbm.at[pl.ds(wid * rows_per_w, rows_per_w)], idxf_v)
        pltpu.sync_copy(w_hbm.at[pl.ds(wid * rows_per_w, rows_per_w)], wf_v)

        def chunk_body(g, carry):
            pltpu.async_copy(
                table_hbm.at[idxf_v.at[pl.ds(g * chunk_rows, chunk_rows)]],
                rows_v, sem,
            ).wait()

            def group_body(g16, c):
                base16 = g16 * 16
                wvec = wf_v[pl.ds(g * chunk_rows + base16, 16)]
                for j in range(16):
                    wb = lax.broadcast(wvec[j], (16,))
                    i = base16 + j
                    rows_v[i, 0:16] = rows_v[i, 0:16] * wb
                    rows_v[i, 16:32] = rows_v[i, 16:32] * wb
                return c

            lax.fori_loop(0, chunk_rows // 16, group_body, 0)

            def out_body(br, c):
                pltpu.sync_copy(
                    rows_v.at[pl.ds(br * n_l, n_l), :],
                    out_hbm.at[b0_out + g * _BC + br],
                )
                return c

            lax.fori_loop(0, _BC, out_body, 0)
            return carry

        lax.fori_loop(0, n_chunks, chunk_body, 0)

    return k(table, idxf, wf)


def kernel(target_indices, target_weights, embedding_weight):
    b, l = target_indices.shape
    bpw = b // _NW
    i = jnp.arange(b * l, dtype=jnp.int32)
    bpos = (i // l + _ROT * bpw) % b
    lpos = i % l
    idxf = target_indices.astype(jnp.int32)[bpos, lpos]
    wf = target_weights[bpos, lpos]
    return _gather_weight(embedding_weight, idxf, wf, b, l)
